# baseline (speedup 1.0000x reference)
import functools

import jax
import jax.numpy as jnp
from jax import lax
from jax.experimental import pallas as pl
from jax.experimental.pallas import tpu as pltpu

EPS = 1e-5
NEG_SLOPE = 0.01


def _fused_kernel(n_chunks, rows_per_chunk, rows_per_step, n_slots, nl,
                  x_hbm, w_ref, gamma_ref, beta_ref, o_hbm,
                  x_vmem, ws_ref, shift_ref, obuf, xsems, osems):
    i = pl.program_id(0)
    n_steps = pl.num_programs(0)
    cin = x_vmem.shape[1]

    @pl.when(i == 0)
    def _stats():
        for c in range(n_chunks):
            sl = pl.ds(c * rows_per_chunk, rows_per_chunk)
            pltpu.make_async_copy(x_hbm.at[sl], x_vmem.at[sl],
                                  xsems.at[c]).start()
        g = jnp.zeros((cin, cin), jnp.float32)
        s = jnp.zeros((cin, 1), jnp.float32)
        for c in range(n_chunks):
            sl = pl.ds(c * rows_per_chunk, rows_per_chunk)
            pltpu.make_async_copy(x_hbm.at[sl], x_vmem.at[sl],
                                  xsems.at[c]).wait()
            for r in range(rows_per_chunk):
                xn = x_vmem[c * rows_per_chunk + r]
                g += lax.dot_general(xn, xn, (((1,), (1,)), ((), ())),
                                     preferred_element_type=jnp.float32)
                s += jnp.sum(xn.astype(jnp.float32), axis=1, keepdims=True)
        inv_nl = jnp.float32(1.0 / nl)
        mean = s * inv_nl
        w32 = w_ref[...]
        mean_y = jnp.dot(w32, mean, preferred_element_type=jnp.float32)
        e_yy = jnp.sum(
            jnp.dot(w32, g * inv_nl, preferred_element_type=jnp.float32) * w32,
            axis=1, keepdims=True)
        var_y = e_yy - mean_y * mean_y
        inv_std = lax.rsqrt(var_y + EPS)
        scale = gamma_ref[...] * inv_std
        ws_ref[...] = w32 * scale
        shift_ref[...] = beta_ref[...] - mean_y * scale

    shift = shift_ref[...]
    ws = ws_ref[...]
    base = i * rows_per_step
    slot = lax.rem(i, n_slots)

    @pl.when(i >= n_slots)
    def _reclaim():
        pltpu.make_async_copy(obuf.at[slot], obuf.at[slot],
                              osems.at[slot]).wait()

    for r in range(rows_per_step):
        z = jnp.dot(ws, x_vmem[base + r],
                    preferred_element_type=jnp.float32) + shift
        obuf[slot, r] = jnp.maximum(z, NEG_SLOPE * z).astype(obuf.dtype)

    pltpu.make_async_copy(obuf.at[slot],
                          o_hbm.at[pl.ds(base, rows_per_step)],
                          osems.at[slot]).start()

    @pl.when(i == n_steps - 1)
    def _drain():
        for s_ in range(n_slots):
            pltpu.make_async_copy(obuf.at[s_], obuf.at[s_],
                                  osems.at[s_]).wait()


def kernel(x, w, b, gamma, beta):
    del b
    N, Cin, L = x.shape
    Cout = w.shape[0]

    rows_per_chunk = next(c for c in (8, 4, 2, 1) if N % c == 0)
    n_chunks = N // rows_per_chunk
    rows_per_step = next(c for c in (4, 2, 1) if N % c == 0)
    n_steps = N // rows_per_step
    n_slots = min(4, n_steps)

    w32 = w.astype(jnp.float32)
    gamma2 = gamma.astype(jnp.float32).reshape(Cout, 1)
    beta2 = beta.astype(jnp.float32).reshape(Cout, 1)

    body = functools.partial(_fused_kernel, n_chunks, rows_per_chunk,
                             rows_per_step, n_slots, N * L)
    return pl.pallas_call(
        body,
        out_shape=jax.ShapeDtypeStruct((N, Cout, L), x.dtype),
        grid=(n_steps,),
        in_specs=[
            pl.BlockSpec(memory_space=pl.ANY),
            pl.BlockSpec((Cout, Cin), lambda i: (0, 0)),
            pl.BlockSpec((Cout, 1), lambda i: (0, 0)),
            pl.BlockSpec((Cout, 1), lambda i: (0, 0)),
        ],
        out_specs=pl.BlockSpec(memory_space=pl.ANY),
        scratch_shapes=[
            pltpu.VMEM((N, Cin, L), x.dtype),
            pltpu.VMEM((Cout, Cin), jnp.float32),
            pltpu.VMEM((Cout, 1), jnp.float32),
            pltpu.VMEM((n_slots, rows_per_step, Cout, L), x.dtype),
            pltpu.SemaphoreType.DMA((n_chunks,)),
            pltpu.SemaphoreType.DMA((n_slots,)),
        ],
        compiler_params=pltpu.CompilerParams(
            dimension_semantics=("arbitrary",),
            vmem_limit_bytes=int(58 << 20),
        ),
    )(x, w32, gamma2, beta2)
